# Initial kernel scaffold; baseline (speedup 1.0000x reference)
import jax
import jax.numpy as jnp
from jax import lax
from jax.experimental import pallas as pl
from jax.experimental.pallas import tpu as pltpu
from jax.experimental.pallas import tpu_sc as plsc

N = 10000
E = 320000
G = 16
CAT = 32
F = 64

NC = 2
NS = 16
NW = NC * NS
EPW = E // NW
CHUNK = 80
NCHUNK = EPW // CHUNK
NBUF = 5
RPT = N // NS

_SC_MESH = plsc.VectorSubcoreMesh(
    core_axis_name="c", subcore_axis_name="s", num_cores=NC, num_subcores=NS)


def _deg_body(dst_hbm, zeros_hbm, ones_hbm, out_hbm, dst_v, ones_v, deg_sh):
    cid = lax.axis_index("c")
    sid = lax.axis_index("s")
    wid = cid * NS + sid
    pltpu.sync_copy(dst_hbm.at[wid], dst_v)
    pltpu.sync_copy(ones_hbm, ones_v)
    pltpu.sync_copy(zeros_hbm.at[pl.ds(sid * RPT, RPT)],
                    deg_sh.at[pl.ds(sid * RPT, RPT)])
    plsc.subcore_barrier()

    def body(j, carry):
        pltpu.sync_copy(ones_v, deg_sh.at[dst_v.at[j]], add=True)
        return carry

    lax.fori_loop(0, NCHUNK, body, 0)
    plsc.subcore_barrier()
    pltpu.sync_copy(deg_sh.at[pl.ds(sid * RPT, RPT)],
                    out_hbm.at[cid, pl.ds(sid * RPT, RPT)])


_deg_kernel = pl.kernel(
    _deg_body,
    out_type=jax.ShapeDtypeStruct((NC, N, 16), jnp.float32),
    mesh=_SC_MESH,
    scratch_types=[
        pltpu.VMEM((NCHUNK, CHUNK), jnp.int32),
        pltpu.VMEM((CHUNK, 16), jnp.float32),
        pltpu.VMEM_SHARED((N, 16), jnp.float32),
    ],
)


def _agg_body(y_hbm, src_hbm, dst_hbm, out_hbm, src_v, dst_v, rows_v, acc_sh,
              sems):
    cid = lax.axis_index("c")
    sid = lax.axis_index("s")
    wid = cid * NS + sid
    pltpu.sync_copy(src_hbm.at[wid], src_v)
    pltpu.sync_copy(dst_hbm.at[wid], dst_v)
    for b in range(NBUF):
        pltpu.async_copy(y_hbm.at[src_v.at[b]], rows_v.at[b], sems.at[b])
    pltpu.sync_copy(y_hbm.at[pl.ds(sid * RPT, RPT)],
                    acc_sh.at[pl.ds(sid * RPT, RPT)])
    plsc.subcore_barrier()

    def group(g, carry):
        for b in range(NBUF):
            jj = g * NBUF + b
            pltpu.make_async_copy(
                y_hbm.at[src_v.at[jj]], rows_v.at[b], sems.at[b]).wait()
            pltpu.sync_copy(rows_v.at[b], acc_sh.at[dst_v.at[jj]], add=True)
            nxt = jj + NBUF

            @pl.when(nxt < NCHUNK)
            def _():
                pltpu.async_copy(
                    y_hbm.at[src_v.at[nxt]], rows_v.at[b], sems.at[b])
        return carry

    lax.fori_loop(0, NCHUNK // NBUF, group, 0)
    plsc.subcore_barrier()
    pltpu.sync_copy(acc_sh.at[pl.ds(sid * RPT, RPT)],
                    out_hbm.at[cid, pl.ds(sid * RPT, RPT)])


_agg_kernel = pl.kernel(
    _agg_body,
    out_type=jax.ShapeDtypeStruct((NC, N, F), jnp.float32),
    mesh=_SC_MESH,
    scratch_types=[
        pltpu.VMEM((NCHUNK, CHUNK), jnp.int32),
        pltpu.VMEM((NCHUNK, CHUNK), jnp.int32),
        pltpu.VMEM((NBUF, CHUNK, F), jnp.float32),
        pltpu.VMEM_SHARED((N, F), jnp.float32),
        pltpu.SemaphoreType.DMA((NBUF,)),
    ],
)


def _mmT(a, w):
    return lax.dot_general(a, w, (((1,), (1,)), ((), ())),
                           preferred_element_type=jnp.float32)


def _dinv(deg_ref):
    deg = deg_ref[0, :, 0:1] + deg_ref[1, :, 0:1] + 1.0
    return lax.rsqrt(deg)


def _elu(h):
    neg = jnp.where(h > 0, 0.0, h)
    return jnp.where(h > 0, h, jnp.expm1(neg))


def _front_body(x_ref, deg_ref, Wc_ref, bc_ref, Wi_ref, bi_ref, W1_ref,
                y_ref):
    dinv = _dinv(deg_ref)
    x = x_ref[...]
    s = jnp.maximum(_mmT(x[:, :CAT], Wc_ref[...]) + bc_ref[...], 0.0)
    h = (_mmT(x[:, CAT:], Wi_ref[...][:, :-CAT])
         + _mmT(s, Wi_ref[...][:, -CAT:]) + bi_ref[...])
    h = jnp.maximum(h, 0.0)
    y_ref[...] = dinv * _mmT(h, W1_ref[...])


_front = pl.pallas_call(
    _front_body,
    out_shape=jax.ShapeDtypeStruct((N, F), jnp.float32),
)


def _mid_body(deg_ref, acc_ref, y_ref, b1_ref, W2_ref, out_ref):
    dinv = _dinv(deg_ref)
    agg = acc_ref[0] + acc_ref[1] - y_ref[...]
    h = _elu(dinv * agg + b1_ref[...])
    out_ref[...] = dinv * _mmT(h, W2_ref[...])


_mid = pl.pallas_call(
    _mid_body,
    out_shape=jax.ShapeDtypeStruct((N, F), jnp.float32),
)


def _back_body(deg_ref, acc_ref, y_ref, batch_ref, b2_ref, Wo_ref, bo_ref,
               out_ref):
    dinv = _dinv(deg_ref)
    agg = acc_ref[0] + acc_ref[1] - y_ref[...]
    h = _elu(dinv * agg + b2_ref[...])
    onehot = (batch_ref[...] ==
              lax.broadcasted_iota(jnp.int32, (1, G), 1)).astype(jnp.float32)
    pooled = lax.dot_general(onehot, h, (((0,), (0,)), ((), ())),
                             preferred_element_type=jnp.float32)
    cnt = lax.dot_general(onehot, jnp.ones((N, 1), jnp.float32),
                          (((0,), (0,)), ((), ())),
                          preferred_element_type=jnp.float32)
    pooled = pooled / jnp.maximum(cnt, 1.0)
    out_ref[...] = _mmT(pooled, Wo_ref[...]) + bo_ref[...]


_back = pl.pallas_call(
    _back_body,
    out_shape=jax.ShapeDtypeStruct((G, F), jnp.float32),
)


def kernel(x, edge_index, batch, Wc, bc, Wi, bi, W1, b1, W2, b2, Wo, bo):
    ei = edge_index.astype(jnp.int32)
    src3 = ei[0].reshape(NW, NCHUNK, CHUNK)
    dst3 = ei[1].reshape(NW, NCHUNK, CHUNK)
    zeros16 = jnp.zeros((N, 16), jnp.float32)
    ones16 = jnp.ones((CHUNK, 16), jnp.float32)

    degp = _deg_kernel(dst3, zeros16, ones16)
    y1 = _front(x, degp, Wc, bc.reshape(1, -1), Wi, bi.reshape(1, -1), W1)
    acc1 = _agg_kernel(y1, src3, dst3)
    y2 = _mid(degp, acc1, y1, b1.reshape(1, -1), W2)
    acc2 = _agg_kernel(y2, src3, dst3)
    return _back(degp, acc2, y2, batch.astype(jnp.int32).reshape(N, 1),
                 b2.reshape(1, -1), Wo, bo.reshape(1, -1))


# speedup vs baseline: 40.3227x; 40.3227x over previous
import jax
import jax.numpy as jnp
from jax import lax
from jax.experimental import pallas as pl
from jax.experimental.pallas import tpu as pltpu
from jax.experimental.pallas import tpu_sc as plsc

N = 10000
E = 320000
G = 16
CAT = 32
F = 64

NC = 2
NS = 16
NW = NC * NS
EPW = E // NW
CHUNK = 80
NCHUNK = EPW // CHUNK
NBUF = 5
NP = 10240
RPT = NP // NS

_SC_MESH = plsc.VectorSubcoreMesh(
    core_axis_name="c", subcore_axis_name="s", num_cores=NC, num_subcores=NS)


def _deg_body(dst_hbm, zeros_hbm, ones_hbm, out_hbm, dst_v, ones_v, deg_sh):
    cid = lax.axis_index("c")
    sid = lax.axis_index("s")
    wid = cid * NS + sid
    pltpu.sync_copy(dst_hbm.at[wid], dst_v)
    pltpu.sync_copy(ones_hbm, ones_v)
    pltpu.sync_copy(zeros_hbm.at[pl.ds(sid * RPT, RPT)],
                    deg_sh.at[pl.ds(sid * RPT, RPT)])
    plsc.subcore_barrier()

    def body(j, carry):
        pltpu.sync_copy(ones_v, deg_sh.at[dst_v.at[j]], add=True)
        return carry

    lax.fori_loop(0, NCHUNK, body, 0)
    plsc.subcore_barrier()
    pltpu.sync_copy(deg_sh.at[pl.ds(sid * RPT, RPT)],
                    out_hbm.at[cid, pl.ds(sid * RPT, RPT)])


_SC_PARAMS = pltpu.CompilerParams(use_tc_tiling_on_sc=False)

_deg_kernel = pl.kernel(
    _deg_body,
    out_type=jax.ShapeDtypeStruct((NC, NP, 16), jnp.float32),
    mesh=_SC_MESH,
    compiler_params=_SC_PARAMS,
    scratch_types=[
        pltpu.VMEM((NCHUNK, CHUNK), jnp.int32),
        pltpu.VMEM((CHUNK, 16), jnp.float32),
        pltpu.VMEM_SHARED((NP, 16), jnp.float32),
    ],
)


def _agg_body(y_hbm, src_hbm, dst_hbm, out_hbm, src_v, dst_v, rows_v, acc_sh,
              sems):
    cid = lax.axis_index("c")
    sid = lax.axis_index("s")
    wid = cid * NS + sid
    pltpu.sync_copy(src_hbm.at[wid], src_v)
    pltpu.sync_copy(dst_hbm.at[wid], dst_v)
    for b in range(NBUF):
        pltpu.async_copy(y_hbm.at[src_v.at[b]], rows_v.at[b], sems.at[b])
    pltpu.sync_copy(y_hbm.at[pl.ds(sid * RPT, RPT)],
                    acc_sh.at[pl.ds(sid * RPT, RPT)])
    plsc.subcore_barrier()

    def group(g, carry):
        for b in range(NBUF):
            jj = g * NBUF + b
            pltpu.make_async_copy(
                y_hbm.at[src_v.at[jj]], rows_v.at[b], sems.at[b]).wait()
            pltpu.sync_copy(rows_v.at[b], acc_sh.at[dst_v.at[jj]], add=True)
            nxt = jj + NBUF

            @pl.when(nxt < NCHUNK)
            def _():
                pltpu.async_copy(
                    y_hbm.at[src_v.at[nxt]], rows_v.at[b], sems.at[b])
        return carry

    lax.fori_loop(0, NCHUNK // NBUF, group, 0)
    plsc.subcore_barrier()
    pltpu.sync_copy(acc_sh.at[pl.ds(sid * RPT, RPT)],
                    out_hbm.at[cid, pl.ds(sid * RPT, RPT)])


_agg_kernel = pl.kernel(
    _agg_body,
    out_type=jax.ShapeDtypeStruct((NC, NP, F), jnp.float32),
    mesh=_SC_MESH,
    compiler_params=_SC_PARAMS,
    scratch_types=[
        pltpu.VMEM((NCHUNK, CHUNK), jnp.int32),
        pltpu.VMEM((NCHUNK, CHUNK), jnp.int32),
        pltpu.VMEM((NBUF, CHUNK, F), jnp.float32),
        pltpu.VMEM_SHARED((NP, F), jnp.float32),
        pltpu.SemaphoreType.DMA((NBUF,)),
    ],
)


def _mmT(a, w):
    return lax.dot_general(a, w, (((1,), (1,)), ((), ())),
                           preferred_element_type=jnp.float32)


def _dinv(deg_ref):
    deg = deg_ref[0, :N, 0:1] + deg_ref[1, :N, 0:1] + 1.0
    return lax.rsqrt(deg)


def _elu(h):
    neg = jnp.where(h > 0, 0.0, h)
    return jnp.where(h > 0, h, jnp.exp(neg) - 1.0)


def _front_body(x_ref, deg_ref, Wc_ref, bc_ref, Wi_ref, bi_ref, W1_ref,
                y_ref):
    dinv = _dinv(deg_ref)
    x = x_ref[...]
    s = jnp.maximum(_mmT(x[:, :CAT], Wc_ref[...]) + bc_ref[...], 0.0)
    h = (_mmT(x[:, CAT:], Wi_ref[...][:, :-CAT])
         + _mmT(s, Wi_ref[...][:, -CAT:]) + bi_ref[...])
    h = jnp.maximum(h, 0.0)
    y_ref[:N] = dinv * _mmT(h, W1_ref[...])
    y_ref[N:] = jnp.zeros((NP - N, F), jnp.float32)


_front = pl.pallas_call(
    _front_body,
    out_shape=jax.ShapeDtypeStruct((NP, F), jnp.float32),
)


def _mid_body(deg_ref, acc_ref, y_ref, b1_ref, W2_ref, out_ref):
    dinv = _dinv(deg_ref)
    agg = acc_ref[0, :N] + acc_ref[1, :N] - y_ref[:N]
    h = _elu(dinv * agg + b1_ref[...])
    out_ref[:N] = dinv * _mmT(h, W2_ref[...])
    out_ref[N:] = jnp.zeros((NP - N, F), jnp.float32)


_mid = pl.pallas_call(
    _mid_body,
    out_shape=jax.ShapeDtypeStruct((NP, F), jnp.float32),
)


def _back_body(deg_ref, acc_ref, y_ref, batch_ref, b2_ref, Wo_ref, bo_ref,
               out_ref):
    dinv = _dinv(deg_ref)
    agg = acc_ref[0, :N] + acc_ref[1, :N] - y_ref[:N]
    h = _elu(dinv * agg + b2_ref[...])
    onehot = (batch_ref[...] ==
              lax.broadcasted_iota(jnp.int32, (1, G), 1)).astype(jnp.float32)
    pooled = lax.dot_general(onehot, h, (((0,), (0,)), ((), ())),
                             preferred_element_type=jnp.float32)
    cnt = lax.dot_general(onehot, jnp.ones((N, 1), jnp.float32),
                          (((0,), (0,)), ((), ())),
                          preferred_element_type=jnp.float32)
    pooled = pooled / jnp.maximum(cnt, 1.0)
    out_ref[...] = _mmT(pooled, Wo_ref[...]) + bo_ref[...]


_back = pl.pallas_call(
    _back_body,
    out_shape=jax.ShapeDtypeStruct((G, F), jnp.float32),
)


def kernel(x, edge_index, batch, Wc, bc, Wi, bi, W1, b1, W2, b2, Wo, bo):
    ei = edge_index.astype(jnp.int32)
    src3 = ei[0].reshape(NW, NCHUNK, CHUNK)
    dst3 = ei[1].reshape(NW, NCHUNK, CHUNK)
    zeros16 = jnp.zeros((NP, 16), jnp.float32)
    ones16 = jnp.ones((CHUNK, 16), jnp.float32)

    degp = _deg_kernel(dst3, zeros16, ones16)
    y1 = _front(x, degp, Wc, bc.reshape(1, -1), Wi, bi.reshape(1, -1), W1)
    acc1 = _agg_kernel(y1, src3, dst3)
    y2 = _mid(degp, acc1, y1, b1.reshape(1, -1), W2)
    acc2 = _agg_kernel(y2, src3, dst3)
    return _back(degp, acc2, y2, batch.astype(jnp.int32).reshape(N, 1),
                 b2.reshape(1, -1), Wo, bo.reshape(1, -1))
